# baseline (speedup 1.0000x reference)
import functools

import jax
import jax.numpy as jnp
from jax import lax
from jax.experimental import pallas as pl
from jax.experimental.pallas import tpu as pltpu
from jax.experimental.pallas import tpu_sc as plsc

EMB = 128
NF = 9
BATCH = 4096
NHALF = 2
HALF = BATCH // NHALF
PAIRS = [(i, j) for i in range(NF) for j in range(i, NF)]
NPAIR = len(PAIRS)
NPLANE = NF + NPAIR + 9 + 1
NC, NS, LANES = 2, 16, 16
NW = NC * NS
BPW = HALF // NW
D1 = 1024
NDENSE = 256

_MESH = plsc.VectorSubcoreMesh(
    core_axis_name="c", subcore_axis_name="s", num_cores=NC, num_subcores=NS)

_NB = 3


def _sc_body(*refs):
    xt = refs[0]
    e1 = refs[1:1 + NF]
    ef = refs[1 + NF:1 + NF + NPAIR]
    es = refs[1 + NF + NPAIR:1 + NF + 2 * NPAIR]
    e3 = refs[1 + NF + 2 * NPAIR:1 + NF + 2 * NPAIR + 9]
    tt = refs[1 + NF + 2 * NPAIR + 9]
    g = refs[1 + NF + 2 * NPAIR + 10]
    idx_v, bufa, bufb, acc_v, sem_i, sem_o = refs[1 + NF + 2 * NPAIR + 11:]

    wid = lax.axis_index("s") * NC + lax.axis_index("c")
    base = wid * BPW
    pltpu.sync_copy(xt.at[:, pl.ds((wid // 2) * 128, 128)], idx_v)
    ioff = (wid % 2) * BPW

    jobs = []
    for i in range(NF):
        jobs.append((e1[i], i, None, None, i))
    for p, (i, j) in enumerate(PAIRS):
        jobs.append((ef[p], i, es[p], j, NF + p))
    for i in range(9):
        jobs.append((e3[i], NF + i, None, None, NF + NPAIR + i))

    def issue(jx):
        t, c, t2, c2, _ = jobs[jx]
        slot = jx % _NB
        d1 = pltpu.async_copy(t.at[idx_v.at[c, pl.ds(ioff, BPW)]], bufa.at[slot], sem_i)
        d2 = None
        if t2 is not None:
            d2 = pltpu.async_copy(t2.at[idx_v.at[c2, pl.ds(ioff, BPW)]], bufb.at[slot], sem_i)
        return d1, d2

    pending = {0: issue(0)}
    writes = []
    for jx in range(len(jobs)):
        _, _, t2, _, plane = jobs[jx]
        slot = jx % _NB
        d1, d2 = pending.pop(jx)
        d1.wait()
        if d2 is not None:
            d2.wait()
        if jx + 1 < len(jobs):
            if len(writes) >= _NB - 1:
                writes.pop(0).wait()
            pending[jx + 1] = issue(jx + 1)
        if t2 is not None:
            def mul_body(r, carry, _slot=slot):
                for u in range(2):
                    for v in range(EMB // LANES):
                        sl = pl.ds(v * LANES, LANES)
                        bufa[_slot, 2 * r + u, sl] = (
                            bufa[_slot, 2 * r + u, sl] * bufb[_slot, 2 * r + u, sl])
                return carry
            lax.fori_loop(0, BPW // 2, mul_body, 0)
        writes.append(pltpu.async_copy(
            bufa.at[slot], g.at[pl.ds(plane * HALF + base, BPW)], sem_o))
    for w in writes:
        w.wait()

    d_acc = pltpu.async_copy(tt.at[idx_v.at[18, pl.ds(ioff, BPW)]], acc_v, sem_i)
    d_acc.wait()
    d_next = pltpu.async_copy(tt.at[idx_v.at[19, pl.ds(ioff, BPW)]], bufa.at[0], sem_i)
    for k in range(1, 10):
        d_next.wait()
        cur = (k - 1) % 2
        if k < 9:
            d_next = pltpu.async_copy(
                tt.at[idx_v.at[18 + k + 1, pl.ds(ioff, BPW)]], bufa.at[k % 2], sem_i)
        def add_body(r, carry, _cur=cur):
            for u in range(2):
                for v in range(EMB // LANES):
                    sl = pl.ds(v * LANES, LANES)
                    acc_v[2 * r + u, sl] = (
                        acc_v[2 * r + u, sl] + bufa[_cur, 2 * r + u, sl])
            return carry
        lax.fori_loop(0, BPW // 2, add_body, 0)
    pltpu.sync_copy(acc_v, g.at[pl.ds((NPLANE - 1) * HALF + base, BPW)])


_sc_gather = functools.partial(
    pl.kernel,
    out_type=jax.ShapeDtypeStruct((NPLANE * HALF, EMB), jnp.float32),
    mesh=_MESH,
    scratch_types=[
        pltpu.VMEM((28, 128), jnp.int32),
        pltpu.VMEM((_NB, BPW, EMB), jnp.float32),
        pltpu.VMEM((_NB, BPW, EMB), jnp.float32),
        pltpu.VMEM((BPW, EMB), jnp.float32),
        pltpu.SemaphoreType.DMA,
        pltpu.SemaphoreType.DMA,
    ],
)(_sc_body)


BM = 1024
BT = HALF // BM
KP = 16
KT = NPLANE // KP


def _l1_body(g_ref, xd_ref, we_ref, wd_ref, b_ref, y_ref, s_ref, q_ref, acc_ref):
    kt = pl.program_id(1)

    @pl.when(kt == 0)
    def _():
        xd = xd_ref[...].astype(jnp.float32)
        acc_ref[...] = jnp.dot(
            xd, wd_ref[...], preferred_element_type=jnp.float32) + b_ref[...]

    part = None
    for t in range(KP):
        d = jnp.dot(g_ref[t].astype(jnp.bfloat16), we_ref[t],
                    preferred_element_type=jnp.float32)
        part = d if part is None else part + d
    acc_ref[...] += part

    @pl.when(kt == KT - 1)
    def _():
        y = acc_ref[...]
        y_ref[...] = y
        s_ref[...] = jnp.sum(y, axis=0, keepdims=True)[None]
        q_ref[...] = jnp.sum(y * y, axis=0, keepdims=True)[None]


def _layer1(g3, xd, we, wd, b1):
    return pl.pallas_call(
        _l1_body,
        grid=(BT, KT),
        in_specs=[
            pl.BlockSpec((KP, BM, EMB), lambda i, k: (k, i, 0)),
            pl.BlockSpec((BM, NDENSE), lambda i, k: (i, 0)),
            pl.BlockSpec((KP, EMB, D1), lambda i, k: (k, 0, 0)),
            pl.BlockSpec((NDENSE, D1), lambda i, k: (0, 0)),
            pl.BlockSpec((1, D1), lambda i, k: (0, 0)),
        ],
        out_specs=[
            pl.BlockSpec((BM, D1), lambda i, k: (i, 0)),
            pl.BlockSpec((1, 1, D1), lambda i, k: (i, 0, 0)),
            pl.BlockSpec((1, 1, D1), lambda i, k: (i, 0, 0)),
        ],
        out_shape=[
            jax.ShapeDtypeStruct((HALF, D1), jnp.float32),
            jax.ShapeDtypeStruct((BT, 1, D1), jnp.float32),
            jax.ShapeDtypeStruct((BT, 1, D1), jnp.float32),
        ],
        scratch_shapes=[pltpu.VMEM((BM, D1), jnp.float32)],
        compiler_params=pltpu.CompilerParams(
            dimension_semantics=("parallel", "arbitrary")),
    )(g3, xd, we, wd, b1)


NPT = NHALF * BT


def _mid_body(y_ref, s_ref, q_ref, w_ref, b_ref, gm_ref, bb_ref,
              y2_ref, s2_ref, q2_ref):
    m = jnp.sum(s_ref[...], axis=0) * (1.0 / BATCH)
    ex2 = jnp.sum(q_ref[...], axis=0) * (1.0 / BATCH)
    inv = 1.0 / jnp.sqrt(ex2 - m * m + 1e-5)
    h = (y_ref[...] - m) * (inv * gm_ref[...]) + bb_ref[...]
    y2 = jnp.dot(h.astype(jnp.bfloat16), w_ref[...].astype(jnp.bfloat16),
                 preferred_element_type=jnp.float32) + b_ref[...]
    y2_ref[...] = y2
    s2_ref[...] = jnp.sum(y2, axis=0, keepdims=True)[None]
    q2_ref[...] = jnp.sum(y2 * y2, axis=0, keepdims=True)[None]


def _mid_layer(y, s, q, w, b, gm, bb):
    din, dout = w.shape
    return pl.pallas_call(
        _mid_body,
        grid=(BT,),
        in_specs=[
            pl.BlockSpec((BM, din), lambda i: (i, 0)),
            pl.BlockSpec((NPT, 1, din), lambda i: (0, 0, 0)),
            pl.BlockSpec((NPT, 1, din), lambda i: (0, 0, 0)),
            pl.BlockSpec((din, dout), lambda i: (0, 0)),
            pl.BlockSpec((1, dout), lambda i: (0, 0)),
            pl.BlockSpec((1, din), lambda i: (0, 0)),
            pl.BlockSpec((1, din), lambda i: (0, 0)),
        ],
        out_specs=[
            pl.BlockSpec((BM, dout), lambda i: (i, 0)),
            pl.BlockSpec((1, 1, dout), lambda i: (i, 0, 0)),
            pl.BlockSpec((1, 1, dout), lambda i: (i, 0, 0)),
        ],
        out_shape=[
            jax.ShapeDtypeStruct((HALF, dout), jnp.float32),
            jax.ShapeDtypeStruct((BT, 1, dout), jnp.float32),
            jax.ShapeDtypeStruct((BT, 1, dout), jnp.float32),
        ],
        compiler_params=pltpu.CompilerParams(
            dimension_semantics=("arbitrary",)),
    )(y, s, q, w, b, gm, bb)


def _fin_body(y_ref, s_ref, q_ref, w_ref, b_ref, gm_ref, bb_ref, o_ref):
    m = jnp.sum(s_ref[...], axis=0) * (1.0 / BATCH)
    ex2 = jnp.sum(q_ref[...], axis=0) * (1.0 / BATCH)
    inv = 1.0 / jnp.sqrt(ex2 - m * m + 1e-5)
    h = (y_ref[...] - m) * (inv * gm_ref[...]) + bb_ref[...]
    o_ref[...] = jax.nn.sigmoid(
        jnp.dot(h, w_ref[...], preferred_element_type=jnp.float32) + b_ref[...])


def _fin_layer(y, s, q, w, b, gm, bb):
    din, dout = w.shape
    return pl.pallas_call(
        _fin_body,
        grid=(BT,),
        in_specs=[
            pl.BlockSpec((BM, din), lambda i: (i, 0)),
            pl.BlockSpec((NPT, 1, din), lambda i: (0, 0, 0)),
            pl.BlockSpec((NPT, 1, din), lambda i: (0, 0, 0)),
            pl.BlockSpec((din, dout), lambda i: (0, 0)),
            pl.BlockSpec((1, dout), lambda i: (0, 0)),
            pl.BlockSpec((1, din), lambda i: (0, 0)),
            pl.BlockSpec((1, din), lambda i: (0, 0)),
        ],
        out_specs=pl.BlockSpec((BM, dout), lambda i: (i, 0)),
        out_shape=jax.ShapeDtypeStruct((HALF, dout), jnp.float32),
        compiler_params=pltpu.CompilerParams(
            dimension_semantics=("arbitrary",)),
    )(y, s, q, w, b, gm, bb)


def kernel(x, emb1, emb2_first, emb2_second, emb3, title_table,
           Ws, bs, bn_scales, bn_biases):
    x = x.astype(jnp.int32)
    xt = x[:, :28].T
    we = Ws[0][:NPLANE * EMB].reshape(NPLANE, EMB, D1).astype(jnp.bfloat16)
    wd = Ws[0][NPLANE * EMB:]
    b1 = bs[0].reshape(1, -1)

    ys, ss, qs = [], [], []
    for h in range(NHALF):
        xth = xt[:, h * HALF:(h + 1) * HALF]
        g = _sc_gather(xth, *emb1, *emb2_first, *emb2_second, *emb3,
                       title_table)
        g3 = g.reshape(NPLANE, HALF, EMB)
        xdh = x[h * HALF:(h + 1) * HALF, 28:284]
        y1h, s1h, q1h = _layer1(g3, xdh, we, wd, b1)
        ys.append(y1h); ss.append(s1h); qs.append(q1h)
    s1 = jnp.concatenate(ss, axis=0)
    q1 = jnp.concatenate(qs, axis=0)

    w4 = jnp.pad(Ws[3], ((0, 0), (0, EMB - Ws[3].shape[1])))
    b4 = jnp.pad(bs[3], (0, EMB - bs[3].shape[0])).reshape(1, -1)

    y2s, y3s, os_ = [], [], []
    s2s, q2s, s3s, q3s = [], [], [], []
    for h in range(NHALF):
        y2h, s2h, q2h = _mid_layer(ys[h], s1, q1, Ws[1], bs[1].reshape(1, -1),
                                   bn_scales[0].reshape(1, -1),
                                   bn_biases[0].reshape(1, -1))
        y2s.append(y2h); s2s.append(s2h); q2s.append(q2h)
    s2 = jnp.concatenate(s2s, axis=0)
    q2 = jnp.concatenate(q2s, axis=0)
    for h in range(NHALF):
        y3h, s3h, q3h = _mid_layer(y2s[h], s2, q2, Ws[2], bs[2].reshape(1, -1),
                                   bn_scales[1].reshape(1, -1),
                                   bn_biases[1].reshape(1, -1))
        y3s.append(y3h); s3s.append(s3h); q3s.append(q3h)
    s3 = jnp.concatenate(s3s, axis=0)
    q3 = jnp.concatenate(q3s, axis=0)
    for h in range(NHALF):
        oh = _fin_layer(y3s[h], s3, q3, w4, b4,
                        bn_scales[2].reshape(1, -1),
                        bn_biases[2].reshape(1, -1))
        os_.append(oh[:, :1])
    return jnp.concatenate(os_, axis=0)


# speedup vs baseline: 1.1085x; 1.1085x over previous
import functools

import jax
import jax.numpy as jnp
from jax import lax
from jax.experimental import pallas as pl
from jax.experimental.pallas import tpu as pltpu
from jax.experimental.pallas import tpu_sc as plsc

EMB = 128
NF = 9
BATCH = 4096
PAIRS = [(i, j) for i in range(NF) for j in range(i, NF)]
NPAIR = len(PAIRS)
NPLANE = NF + NPAIR + 9 + 1
SPLIT = 32
NC, NS, LANES = 2, 16, 16
NW = NC * NS
BPW = BATCH // NW
D1 = 1024
NDENSE = 256
NPA = 23

_MESH = plsc.VectorSubcoreMesh(
    core_axis_name="c", subcore_axis_name="s", num_cores=NC, num_subcores=NS)

_NB = 3


def _gather_jobs(body_refs, jobs, idx_v, bufa, bufb, g, sem_i, sem_o, base):
    def issue(jx):
        t, c, t2, c2, _ = jobs[jx]
        slot = jx % _NB
        d1 = pltpu.async_copy(t.at[idx_v.at[c]], bufa.at[slot], sem_i)
        d2 = None
        if t2 is not None:
            d2 = pltpu.async_copy(t2.at[idx_v.at[c2]], bufb.at[slot], sem_i)
        return d1, d2

    pending = {0: issue(0)}
    writes = []
    for jx in range(len(jobs)):
        _, _, t2, _, plane = jobs[jx]
        slot = jx % _NB
        d1, d2 = pending.pop(jx)
        d1.wait()
        if d2 is not None:
            d2.wait()
        if jx + 1 < len(jobs):
            if len(writes) >= _NB - 1:
                writes.pop(0).wait()
            pending[jx + 1] = issue(jx + 1)
        if t2 is not None:
            def mul_body(r, carry, _slot=slot):
                for u in range(2):
                    for v in range(EMB // LANES):
                        sl = pl.ds(v * LANES, LANES)
                        bufa[_slot, 2 * r + u, sl] = (
                            bufa[_slot, 2 * r + u, sl]
                            * bufb[_slot, 2 * r + u, sl])
                return carry
            lax.fori_loop(0, BPW // 2, mul_body, 0)
        writes.append(pltpu.async_copy(
            bufa.at[slot], g.at[pl.ds(plane * BATCH + base, BPW)], sem_o))
    for w in writes:
        w.wait()


def _sc_a_body(*refs):
    xt = refs[0]
    e1 = refs[1:1 + NF]
    ef = refs[1 + NF:1 + NF + NPA]
    es = refs[1 + NF + NPA:1 + NF + 2 * NPA]
    g = refs[1 + NF + 2 * NPA]
    idx_v, bufa, bufb, sem_i, sem_o = refs[2 + NF + 2 * NPA:]

    wid = lax.axis_index("s") * NC + lax.axis_index("c")
    base = wid * BPW
    pltpu.sync_copy(xt.at[:, pl.ds(base, BPW)], idx_v)

    jobs = []
    for i in range(NF):
        jobs.append((e1[i], i, None, None, i))
    for p in range(NPA):
        i, j = PAIRS[p]
        jobs.append((ef[p], i, es[p], j, NF + p))
    _gather_jobs(refs, jobs, idx_v, bufa, bufb, g, sem_i, sem_o, base)


def _sc_b_body(*refs):
    npb = NPAIR - NPA
    xt = refs[0]
    ef = refs[1:1 + npb]
    es = refs[1 + npb:1 + 2 * npb]
    e3 = refs[1 + 2 * npb:10 + 2 * npb]
    tt = refs[10 + 2 * npb]
    g = refs[11 + 2 * npb]
    idx_v, bufa, bufb, acc_v, sem_i, sem_o = refs[12 + 2 * npb:]

    wid = lax.axis_index("s") * NC + lax.axis_index("c")
    base = wid * BPW
    pltpu.sync_copy(xt.at[:, pl.ds(base, BPW)], idx_v)

    jobs = []
    for p in range(NPA, NPAIR):
        i, j = PAIRS[p]
        jobs.append((ef[p - NPA], i, es[p - NPA], j, NF + p - SPLIT))
    for i in range(9):
        jobs.append((e3[i], NF + i, None, None, NF + NPAIR + i - SPLIT))
    _gather_jobs(refs, jobs, idx_v, bufa, bufb, g, sem_i, sem_o, base)

    d_acc = pltpu.async_copy(tt.at[idx_v.at[18]], acc_v, sem_i)
    d_acc.wait()
    d_next = pltpu.async_copy(tt.at[idx_v.at[19]], bufa.at[0], sem_i)
    for k in range(1, 10):
        d_next.wait()
        cur = (k - 1) % 2
        if k < 9:
            d_next = pltpu.async_copy(
                tt.at[idx_v.at[18 + k + 1]], bufa.at[k % 2], sem_i)
        def add_body(r, carry, _cur=cur):
            for u in range(2):
                for v in range(EMB // LANES):
                    sl = pl.ds(v * LANES, LANES)
                    acc_v[2 * r + u, sl] = (
                        acc_v[2 * r + u, sl] + bufa[_cur, 2 * r + u, sl])
            return carry
        lax.fori_loop(0, BPW // 2, add_body, 0)
    pltpu.sync_copy(
        acc_v, g.at[pl.ds((NPLANE - 1 - SPLIT) * BATCH + base, BPW)])


def _sc_scratch(with_acc):
    s = [
        pltpu.VMEM((28, BPW), jnp.int32),
        pltpu.VMEM((_NB, BPW, EMB), jnp.float32),
        pltpu.VMEM((_NB, BPW, EMB), jnp.float32),
    ]
    if with_acc:
        s.append(pltpu.VMEM((BPW, EMB), jnp.float32))
    s += [pltpu.SemaphoreType.DMA, pltpu.SemaphoreType.DMA]
    return s


_sc_a = functools.partial(
    pl.kernel,
    out_type=jax.ShapeDtypeStruct((SPLIT * BATCH, EMB), jnp.float32),
    mesh=_MESH,
    scratch_types=_sc_scratch(False),
)(_sc_a_body)

_sc_b = functools.partial(
    pl.kernel,
    out_type=jax.ShapeDtypeStruct(((NPLANE - SPLIT) * BATCH, EMB), jnp.float32),
    mesh=_MESH,
    scratch_types=_sc_scratch(True),
)(_sc_b_body)


BM = 1024
BT = BATCH // BM
KP = 16
KT = SPLIT // KP


def _l1a_body(g_ref, xd_ref, we_ref, wd_ref, b_ref, y_ref, acc_ref):
    kt = pl.program_id(1)

    @pl.when(kt == 0)
    def _():
        xd = xd_ref[...].astype(jnp.float32)
        acc_ref[...] = jnp.dot(
            xd, wd_ref[...], preferred_element_type=jnp.float32) + b_ref[...]

    part = None
    for t in range(KP):
        d = jnp.dot(g_ref[t].astype(jnp.bfloat16), we_ref[t],
                    preferred_element_type=jnp.float32)
        part = d if part is None else part + d
    acc_ref[...] += part

    @pl.when(kt == KT - 1)
    def _():
        y_ref[...] = acc_ref[...]


def _l1a(g3, xd, we, wd, b1):
    return pl.pallas_call(
        _l1a_body,
        grid=(BT, KT),
        in_specs=[
            pl.BlockSpec((KP, BM, EMB), lambda i, k: (k, i, 0)),
            pl.BlockSpec((BM, NDENSE), lambda i, k: (i, 0)),
            pl.BlockSpec((KP, EMB, D1), lambda i, k: (k, 0, 0)),
            pl.BlockSpec((NDENSE, D1), lambda i, k: (0, 0)),
            pl.BlockSpec((1, D1), lambda i, k: (0, 0)),
        ],
        out_specs=pl.BlockSpec((BM, D1), lambda i, k: (i, 0)),
        out_shape=jax.ShapeDtypeStruct((BATCH, D1), jnp.float32),
        scratch_shapes=[pltpu.VMEM((BM, D1), jnp.float32)],
        compiler_params=pltpu.CompilerParams(
            dimension_semantics=("parallel", "arbitrary")),
    )(g3, xd, we, wd, b1)


def _l1b_body(g_ref, yp_ref, we_ref, y_ref, s_ref, q_ref, acc_ref):
    kt = pl.program_id(1)

    @pl.when(kt == 0)
    def _():
        acc_ref[...] = yp_ref[...]

    part = None
    for t in range(KP):
        d = jnp.dot(g_ref[t].astype(jnp.bfloat16), we_ref[t],
                    preferred_element_type=jnp.float32)
        part = d if part is None else part + d
    acc_ref[...] += part

    @pl.when(kt == KT - 1)
    def _():
        y = acc_ref[...]
        y_ref[...] = y
        s_ref[...] = jnp.sum(y, axis=0, keepdims=True)[None]
        q_ref[...] = jnp.sum(y * y, axis=0, keepdims=True)[None]


def _l1b(g3, yp, we):
    return pl.pallas_call(
        _l1b_body,
        grid=(BT, KT),
        in_specs=[
            pl.BlockSpec((KP, BM, EMB), lambda i, k: (k, i, 0)),
            pl.BlockSpec((BM, D1), lambda i, k: (i, 0)),
            pl.BlockSpec((KP, EMB, D1), lambda i, k: (k, 0, 0)),
        ],
        out_specs=[
            pl.BlockSpec((BM, D1), lambda i, k: (i, 0)),
            pl.BlockSpec((1, 1, D1), lambda i, k: (i, 0, 0)),
            pl.BlockSpec((1, 1, D1), lambda i, k: (i, 0, 0)),
        ],
        out_shape=[
            jax.ShapeDtypeStruct((BATCH, D1), jnp.float32),
            jax.ShapeDtypeStruct((BT, 1, D1), jnp.float32),
            jax.ShapeDtypeStruct((BT, 1, D1), jnp.float32),
        ],
        scratch_shapes=[pltpu.VMEM((BM, D1), jnp.float32)],
        compiler_params=pltpu.CompilerParams(
            dimension_semantics=("parallel", "arbitrary")),
    )(g3, yp, we)


def _mid_body(y_ref, s_ref, q_ref, w_ref, b_ref, gm_ref, bb_ref,
              y2_ref, s2_ref, q2_ref):
    m = jnp.sum(s_ref[...], axis=0) * (1.0 / BATCH)
    ex2 = jnp.sum(q_ref[...], axis=0) * (1.0 / BATCH)
    inv = 1.0 / jnp.sqrt(ex2 - m * m + 1e-5)
    h = (y_ref[...] - m) * (inv * gm_ref[...]) + bb_ref[...]
    y2 = jnp.dot(h.astype(jnp.bfloat16), w_ref[...].astype(jnp.bfloat16),
                 preferred_element_type=jnp.float32) + b_ref[...]
    y2_ref[...] = y2
    s2_ref[...] = jnp.sum(y2, axis=0, keepdims=True)[None]
    q2_ref[...] = jnp.sum(y2 * y2, axis=0, keepdims=True)[None]


def _mid_layer(y, s, q, w, b, gm, bb):
    din, dout = w.shape
    return pl.pallas_call(
        _mid_body,
        grid=(BT,),
        in_specs=[
            pl.BlockSpec((BM, din), lambda i: (i, 0)),
            pl.BlockSpec((BT, 1, din), lambda i: (0, 0, 0)),
            pl.BlockSpec((BT, 1, din), lambda i: (0, 0, 0)),
            pl.BlockSpec((din, dout), lambda i: (0, 0)),
            pl.BlockSpec((1, dout), lambda i: (0, 0)),
            pl.BlockSpec((1, din), lambda i: (0, 0)),
            pl.BlockSpec((1, din), lambda i: (0, 0)),
        ],
        out_specs=[
            pl.BlockSpec((BM, dout), lambda i: (i, 0)),
            pl.BlockSpec((1, 1, dout), lambda i: (i, 0, 0)),
            pl.BlockSpec((1, 1, dout), lambda i: (i, 0, 0)),
        ],
        out_shape=[
            jax.ShapeDtypeStruct((BATCH, dout), jnp.float32),
            jax.ShapeDtypeStruct((BT, 1, dout), jnp.float32),
            jax.ShapeDtypeStruct((BT, 1, dout), jnp.float32),
        ],
        compiler_params=pltpu.CompilerParams(
            dimension_semantics=("arbitrary",)),
    )(y, s, q, w, b, gm, bb)


def _fin_body(y_ref, s_ref, q_ref, w_ref, b_ref, gm_ref, bb_ref, o_ref):
    m = jnp.sum(s_ref[...], axis=0) * (1.0 / BATCH)
    ex2 = jnp.sum(q_ref[...], axis=0) * (1.0 / BATCH)
    inv = 1.0 / jnp.sqrt(ex2 - m * m + 1e-5)
    h = (y_ref[...] - m) * (inv * gm_ref[...]) + bb_ref[...]
    o_ref[...] = jax.nn.sigmoid(
        jnp.dot(h, w_ref[...], preferred_element_type=jnp.float32) + b_ref[...])


def _fin_layer(y, s, q, w, b, gm, bb):
    din, dout = w.shape
    return pl.pallas_call(
        _fin_body,
        grid=(BT,),
        in_specs=[
            pl.BlockSpec((BM, din), lambda i: (i, 0)),
            pl.BlockSpec((BT, 1, din), lambda i: (0, 0, 0)),
            pl.BlockSpec((BT, 1, din), lambda i: (0, 0, 0)),
            pl.BlockSpec((din, dout), lambda i: (0, 0)),
            pl.BlockSpec((1, dout), lambda i: (0, 0)),
            pl.BlockSpec((1, din), lambda i: (0, 0)),
            pl.BlockSpec((1, din), lambda i: (0, 0)),
        ],
        out_specs=pl.BlockSpec((BM, dout), lambda i: (i, 0)),
        out_shape=jax.ShapeDtypeStruct((BATCH, dout), jnp.float32),
        compiler_params=pltpu.CompilerParams(
            dimension_semantics=("arbitrary",)),
    )(y, s, q, w, b, gm, bb)


def kernel(x, emb1, emb2_first, emb2_second, emb3, title_table,
           Ws, bs, bn_scales, bn_biases):
    x = x.astype(jnp.int32)
    xt = x[:, :28].T
    xd = x[:, 28:284]
    we = Ws[0][:NPLANE * EMB].reshape(NPLANE, EMB, D1).astype(jnp.bfloat16)
    wd = Ws[0][NPLANE * EMB:]
    b1 = bs[0].reshape(1, -1)

    ga = _sc_a(xt, *emb1, *emb2_first[:NPA], *emb2_second[:NPA])
    gb = _sc_b(xt, *emb2_first[NPA:], *emb2_second[NPA:], *emb3, title_table)

    yp = _l1a(ga.reshape(SPLIT, BATCH, EMB), xd, we[:SPLIT], wd, b1)
    y1, s1, q1 = _l1b(gb.reshape(NPLANE - SPLIT, BATCH, EMB), yp, we[SPLIT:])

    y2, s2, q2 = _mid_layer(y1, s1, q1, Ws[1], bs[1].reshape(1, -1),
                            bn_scales[0].reshape(1, -1),
                            bn_biases[0].reshape(1, -1))
    y3, s3, q3 = _mid_layer(y2, s2, q2, Ws[2], bs[2].reshape(1, -1),
                            bn_scales[1].reshape(1, -1),
                            bn_biases[1].reshape(1, -1))
    w4 = jnp.pad(Ws[3], ((0, 0), (0, EMB - Ws[3].shape[1])))
    b4 = jnp.pad(bs[3], (0, EMB - bs[3].shape[0])).reshape(1, -1)
    o = _fin_layer(y3, s3, q3, w4, b4,
                   bn_scales[2].reshape(1, -1), bn_biases[2].reshape(1, -1))
    return o[:, :1]
